# Initial kernel scaffold; baseline (speedup 1.0000x reference)
import jax
import jax.numpy as jnp
from jax.experimental import pallas as pl


def kernel(location, table):
    raise NotImplementedError("write your pallas kernel here")


# speedup vs baseline: 3.0653x; 3.0653x over previous
import functools

import jax
import jax.numpy as jnp
from jax import lax
from jax.experimental import pallas as pl
from jax.experimental.pallas import tpu as pltpu
from jax.experimental.pallas import tpu_sc as plsc

ROWS = 16384
COLS = 200
D = 64
B = ROWS * COLS

NW = 32
B_PER_W = B // NW
G = 128
GROUPS = 8
CHUNK = G * GROUPS
N_CHUNKS = B_PER_W // CHUNK


def _gather_kernel(idx2d, table):
    mesh = plsc.VectorSubcoreMesh(core_axis_name="c", subcore_axis_name="s")

    @functools.partial(
        pl.kernel,
        mesh=mesh,
        compiler_params=pltpu.CompilerParams(use_tc_tiling_on_sc=False),
        out_type=jax.ShapeDtypeStruct((B, D), jnp.float32),
        scratch_types=[
            pltpu.VMEM((GROUPS, G), jnp.int32),
            pltpu.VMEM((CHUNK, D), jnp.float32),
            pltpu.SemaphoreType.DMA,
        ],
    )
    def k(idx_hbm, table_hbm, out_hbm, idx_v, rows_v, sem):
        wid = lax.axis_index("s") * 2 + lax.axis_index("c")
        base_grp = wid * (B_PER_W // G)

        def body(c, carry):
            grp0 = base_grp + c * GROUPS
            pltpu.sync_copy(idx_hbm.at[pl.ds(grp0, GROUPS)], idx_v)
            handles = [
                pltpu.async_copy(
                    table_hbm.at[idx_v.at[g]],
                    rows_v.at[pl.ds(g * G, G)],
                    sem,
                )
                for g in range(GROUPS)
            ]
            for h in handles:
                h.wait()
            pltpu.sync_copy(rows_v, out_hbm.at[pl.ds(grp0 * G, CHUNK)])
            return carry

        lax.fori_loop(0, N_CHUNKS, body, 0)

    return k(idx2d, table)


def kernel(location, table):
    idx2d = location.reshape(B // G, G).astype(jnp.int32)
    out = _gather_kernel(idx2d, table)
    return out.reshape(ROWS, COLS, D)


# speedup vs baseline: 3.0853x; 1.0065x over previous
import functools

import jax
import jax.numpy as jnp
from jax import lax
from jax.experimental import pallas as pl
from jax.experimental.pallas import tpu as pltpu
from jax.experimental.pallas import tpu_sc as plsc

ROWS = 16384
COLS = 200
D = 64
B = ROWS * COLS

NW = 32
B_PER_W = B // NW
G = 128
GROUPS = 5
CHUNK = G * GROUPS
N_CHUNKS = B_PER_W // CHUNK
GRP_PER_W = B_PER_W // G


def _gather_kernel(idx2d, table):
    mesh = plsc.VectorSubcoreMesh(core_axis_name="c", subcore_axis_name="s")

    @functools.partial(
        pl.kernel,
        mesh=mesh,
        compiler_params=pltpu.CompilerParams(use_tc_tiling_on_sc=False),
        out_type=jax.ShapeDtypeStruct((B, D), jnp.float32),
        scratch_types=[
            pltpu.VMEM((2, GROUPS, G), jnp.int32),
            pltpu.VMEM((2, CHUNK, D), jnp.float32),
            pltpu.SemaphoreType.DMA,
            pltpu.SemaphoreType.DMA,
            pltpu.SemaphoreType.DMA,
            pltpu.SemaphoreType.DMA,
            pltpu.SemaphoreType.DMA,
        ],
    )
    def k(idx_hbm, table_hbm, out_hbm, idx_v, rows_v, sem_g,
          sem_i0, sem_i1, sem_o0, sem_o1):
        wid = lax.axis_index("s") * 2 + lax.axis_index("c")
        base_grp = wid * GRP_PER_W
        sem_i = (sem_i0, sem_i1)
        sem_o = (sem_o0, sem_o1)

        def idx_start(c, b):
            grp0 = jnp.minimum(base_grp + c * GROUPS, B // G - GROUPS)
            return pltpu.make_async_copy(
                idx_hbm.at[pl.ds(grp0, GROUPS)], idx_v.at[b], sem_i[b])

        def gathers(c, b):
            grp0 = base_grp + c * GROUPS
            handles = [
                pltpu.async_copy(
                    table_hbm.at[idx_v.at[b, g]],
                    rows_v.at[b, pl.ds(g * G, G)],
                    sem_g,
                )
                for g in range(GROUPS)
            ]
            for h in handles:
                h.wait()
            return pltpu.make_async_copy(
                rows_v.at[b], out_hbm.at[pl.ds(grp0 * G, CHUNK)], sem_o[b])

        idx_start(0, 0).start()
        idx_start(1, 1).start()

        idx_start(0, 0).wait()
        gathers(0, 0).start()
        idx_start(2, 0).start()
        idx_start(1, 1).wait()
        gathers(1, 1).start()
        idx_start(3, 1).start()

        def step(c, b):
            pltpu.make_async_copy(
                rows_v.at[b], out_hbm.at[pl.ds(0, CHUNK)], sem_o[b]).wait()
            idx_start(c, b).wait()
            gathers(c, b).start()
            idx_start(c + 2, b).start()

        def body(i, carry):
            c = 2 + 2 * i
            step(c, 0)
            step(c + 1, 1)
            return carry

        lax.fori_loop(0, (N_CHUNKS - 2) // 2, body, 0)

        for b in range(2):
            pltpu.make_async_copy(
                rows_v.at[b], out_hbm.at[pl.ds(0, CHUNK)], sem_o[b]).wait()
            idx_start(0, b).wait()

    return k(idx2d, table)


def kernel(location, table):
    idx2d = location.reshape(B // G, G).astype(jnp.int32)
    out = _gather_kernel(idx2d, table)
    return out.reshape(ROWS, COLS, D)


# speedup vs baseline: 5.8163x; 1.8852x over previous
import functools

import jax
import jax.numpy as jnp
from jax import lax
from jax.experimental import pallas as pl
from jax.experimental.pallas import tpu as pltpu
from jax.experimental.pallas import tpu_sc as plsc

ROWS = 16384
COLS = 200
D = 64
B = ROWS * COLS

NW = 32
B_PER_W = B // NW
G = 128
GROUPS = 5
CHUNK = G * GROUPS
N_CHUNKS = B_PER_W // CHUNK
GRP_PER_W = B_PER_W // G


def _gather_kernel(idx2d, table):
    mesh = plsc.VectorSubcoreMesh(core_axis_name="c", subcore_axis_name="s")

    @functools.partial(
        pl.kernel,
        mesh=mesh,
        compiler_params=pltpu.CompilerParams(use_tc_tiling_on_sc=False),
        out_type=jax.ShapeDtypeStruct((B, D), jnp.float32),
        scratch_types=[
            pltpu.VMEM((2, GROUPS, G), jnp.int32),
            pltpu.VMEM((2, CHUNK, D), jnp.float32),
            pltpu.VMEM_SHARED((202, D), jnp.float32),
            pltpu.SemaphoreType.DMA,
            pltpu.SemaphoreType.DMA,
            pltpu.SemaphoreType.DMA,
            pltpu.SemaphoreType.DMA,
            pltpu.SemaphoreType.DMA,
        ],
    )
    def k(idx_hbm, table_hbm, out_hbm, idx_v, rows_v, table_sh, sem_g,
          sem_i0, sem_i1, sem_o0, sem_o1):
        wid = lax.axis_index("s") * 2 + lax.axis_index("c")
        base_grp = wid * GRP_PER_W
        sem_i = (sem_i0, sem_i1)
        sem_o = (sem_o0, sem_o1)

        @pl.when(lax.axis_index("s") == 0)
        def _():
            pltpu.sync_copy(table_hbm, table_sh)

        plsc.subcore_barrier()

        def idx_start(c, b):
            grp0 = jnp.minimum(base_grp + c * GROUPS, B // G - GROUPS)
            return pltpu.make_async_copy(
                idx_hbm.at[pl.ds(grp0, GROUPS)], idx_v.at[b], sem_i[b])

        def gathers(c, b):
            grp0 = base_grp + c * GROUPS
            handles = [
                pltpu.async_copy(
                    table_sh.at[idx_v.at[b, g]],
                    rows_v.at[b, pl.ds(g * G, G)],
                    sem_g,
                )
                for g in range(GROUPS)
            ]
            for h in handles:
                h.wait()
            return pltpu.make_async_copy(
                rows_v.at[b], out_hbm.at[pl.ds(grp0 * G, CHUNK)], sem_o[b])

        idx_start(0, 0).start()
        idx_start(1, 1).start()

        idx_start(0, 0).wait()
        gathers(0, 0).start()
        idx_start(2, 0).start()
        idx_start(1, 1).wait()
        gathers(1, 1).start()
        idx_start(3, 1).start()

        def step(c, b):
            pltpu.make_async_copy(
                rows_v.at[b], out_hbm.at[pl.ds(0, CHUNK)], sem_o[b]).wait()
            idx_start(c, b).wait()
            gathers(c, b).start()
            idx_start(c + 2, b).start()

        def body(i, carry):
            c = 2 + 2 * i
            step(c, 0)
            step(c + 1, 1)
            return carry

        lax.fori_loop(0, (N_CHUNKS - 2) // 2, body, 0)

        for b in range(2):
            pltpu.make_async_copy(
                rows_v.at[b], out_hbm.at[pl.ds(0, CHUNK)], sem_o[b]).wait()
            idx_start(0, b).wait()

    return k(idx2d, table)


def kernel(location, table):
    idx2d = location.reshape(B // G, G).astype(jnp.int32)
    out = _gather_kernel(idx2d, table)
    return out.reshape(ROWS, COLS, D)


# speedup vs baseline: 14.7096x; 2.5290x over previous
import functools

import jax
import jax.numpy as jnp
from jax import lax
from jax.experimental import pallas as pl
from jax.experimental.pallas import tpu as pltpu
from jax.experimental.pallas import tpu_sc as plsc

ROWS = 16384
COLS = 200
D = 64
V = 202
NW = 32
CI = 512
SUB_PER_J = ROWS // CI
N_SUB = COLS * SUB_PER_J
SUB_PER_W = N_SUB // NW
L = 16


def _lookup_kernel(locT, tableT):
    mesh = plsc.VectorSubcoreMesh(core_axis_name="c", subcore_axis_name="s")

    @functools.partial(
        pl.kernel,
        mesh=mesh,
        compiler_params=pltpu.CompilerParams(
            use_tc_tiling_on_sc=False, needs_layout_passes=False),
        out_type=jax.ShapeDtypeStruct((COLS, D, ROWS), jnp.float32),
        scratch_types=[
            pltpu.VMEM((2, CI), jnp.int32),
            pltpu.VMEM((2, D, CI), jnp.float32),
            pltpu.VMEM((D * V,), jnp.float32),
            pltpu.SemaphoreType.DMA,
            pltpu.SemaphoreType.DMA,
            pltpu.SemaphoreType.DMA,
            pltpu.SemaphoreType.DMA,
        ],
    )
    def k(locT_hbm, tblT_hbm, out_hbm, idx_v, outbuf, tbl_v,
          sem_i0, sem_i1, sem_o0, sem_o1):
        wid = lax.axis_index("s") * 2 + lax.axis_index("c")
        sem_i = (sem_i0, sem_i1)
        sem_o = (sem_o0, sem_o1)

        pltpu.sync_copy(tblT_hbm, tbl_v)

        def sub_pos(t):
            s = jnp.minimum(wid * SUB_PER_W + t, N_SUB - 1)
            return s // SUB_PER_J, (s % SUB_PER_J) * CI

        def idx_fetch(t, b):
            j, i0 = sub_pos(t)
            return pltpu.make_async_copy(
                locT_hbm.at[j, pl.ds(i0, CI)], idx_v.at[b], sem_i[b])

        def out_store(t, b):
            j, i0 = sub_pos(t)
            return pltpu.make_async_copy(
                outbuf.at[b], out_hbm.at[j, :, pl.ds(i0, CI)], sem_o[b])

        def compute(b):
            def grp(k16, carry):
                idxreg = idx_v[b, pl.ds(k16 * L, L)]
                for d in range(D):
                    outbuf[b, d, pl.ds(k16 * L, L)] = plsc.load_gather(
                        tbl_v, [idxreg + jnp.int32(d * V)])
                return carry
            lax.fori_loop(0, CI // L, grp, 0)

        def step(t, b, first):
            if not first:
                out_store(t, b).wait()
            idx_fetch(t, b).wait()
            compute(b)
            idx_fetch(t + 2, b).start()
            out_store(t, b).start()

        idx_fetch(0, 0).start()
        idx_fetch(1, 1).start()
        step(0, 0, True)
        step(1, 1, True)

        def body(p, carry):
            t = 2 + 2 * p
            step(t, 0, False)
            step(t + 1, 1, False)
            return carry

        lax.fori_loop(0, (SUB_PER_W - 2) // 2, body, 0)

        for b in range(2):
            out_store(0, b).wait()
            idx_fetch(0, b).wait()

    return k(locT, tableT.reshape(-1))


def kernel(location, table):
    locT = location.transpose().astype(jnp.int32)
    tableT = table.transpose()
    outT = _lookup_kernel(locT, tableT)
    return jnp.transpose(outT, (2, 0, 1))
